# Initial kernel scaffold; baseline (speedup 1.0000x reference)
import jax
import jax.numpy as jnp
from jax.experimental import pallas as pl


def kernel(centroids_coords, corr_pyramid_0, corr_pyramid_1, corr_pyramid_2, corr_pyramid_3):
    raise NotImplementedError("write your pallas kernel here")


# speedup vs baseline: 2.9684x; 2.9684x over previous
import functools

import jax
import jax.numpy as jnp
from jax import lax
from jax.experimental import pallas as pl
from jax.experimental.pallas import tpu as pltpu, tpu_sc as plsc

RADIUS = 4
NUM_LEVELS = 4
K = 2 * RADIUS + 1

NC, NS, L = 2, 16, 16
NW = NC * NS

B, H, W = 8, 128, 160
HW = H * W
N = B * HW
PIX_PER_W = N // NW
CHUNK = 128
NCHUNK = PIX_PER_W // CHUNK
WLS = [W >> l for l in range(NUM_LEVELS)]


def _body(cc_hbm, c0, c1, c2, c3, out_hbm,
          ccbuf, win0, win1, win2, win3,
          r0b0, r1b0, r0b1, r1b1, r0b2, r1b2, r0b3, r1b3,
          outbuf, sem_g, sem_o):
    corrs = [c0, c1, c2, c3]
    wins = [win0, win1, win2, win3]
    r0bufs = [r0b0, r0b1, r0b2, r0b3]
    r1bufs = [r1b0, r1b1, r1b2, r1b3]

    wid = lax.axis_index("s") * NC + lax.axis_index("c")
    base_pix = wid * PIX_PER_W
    b = wid // 4
    hw_base = (wid % 4) * PIX_PER_W

    pltpu.sync_copy(cc_hbm.at[pl.ds(base_pix, PIX_PER_W)], ccbuf)

    lane = lax.iota(jnp.int32, L)

    def chunk_body(jc, carry):
        chunk0 = jc * CHUNK
        for l in range(NUM_LEVELS):
            wl = WLS[l]
            nrows = N * wl // 16
            for g in range(CHUNK // L):
                p_loc = g * L + lane
                cc = ccbuf[pl.ds(chunk0 + g * L, L)]
                c = cc * (0.5 ** l)
                i = c.astype(jnp.int32)
                base = i - RADIUS
                p_glob = base_pix + chunk0 + p_loc
                s = p_glob * wl + base
                r0 = s >> 4
                r0bufs[l][pl.ds(g * L, L)] = jnp.maximum(r0, 0)
                r1bufs[l][pl.ds(g * L, L)] = jnp.minimum(r0 + 1, nrows - 1)
        cps = []
        for l in range(NUM_LEVELS):
            cps.append(pltpu.async_copy(
                corrs[l].at[r0bufs[l]], wins[l].at[pl.ds(0, CHUNK)], sem_g))
            cps.append(pltpu.async_copy(
                corrs[l].at[r1bufs[l]], wins[l].at[pl.ds(CHUNK, CHUNK)], sem_g))
        for cp in cps:
            cp.wait()
        for l in range(NUM_LEVELS):
            wl = WLS[l]
            for g in range(CHUNK // L):
                p_loc = g * L + lane
                cc = ccbuf[pl.ds(chunk0 + g * L, L)]
                c = cc * (0.5 ** l)
                i = c.astype(jnp.int32)
                frac = c - i.astype(jnp.float32)
                omf = 1.0 - frac
                base = i - RADIUS
                p_glob = base_pix + chunk0 + p_loc
                s = p_glob * wl + base
                off = s & 15
                row_a = p_loc
                row_b = p_loc + CHUNK
                vs = []
                for m in range(K + 1):
                    q = off + m
                    ge = q >= 16
                    row = jnp.where(ge, row_b, row_a)
                    col = q & 15
                    v = plsc.load_gather(wins[l], [row, col])
                    bm = base + m
                    valid = (bm >= 0) & (bm < wl)
                    vs.append(jnp.where(valid, v, 0.0))
                for k in range(K):
                    ch = l * K + k
                    outbuf[pl.ds(ch * CHUNK + g * L, L)] = (
                        vs[k] * omf + vs[k + 1] * frac)
        ocps = []
        for ch in range(NUM_LEVELS * K):
            ocps.append(pltpu.async_copy(
                outbuf.at[pl.ds(ch * CHUNK, CHUNK)],
                out_hbm.at[b, ch, pl.ds(hw_base + chunk0, CHUNK)], sem_o))
        for cp in ocps:
            cp.wait()
        return carry

    lax.fori_loop(0, NCHUNK, chunk_body, 0)


@jax.jit
def kernel(centroids_coords, corr_pyramid_0, corr_pyramid_1, corr_pyramid_2,
           corr_pyramid_3):
    cc_flat = centroids_coords[:, 0].reshape(N)
    corr2d = [c.reshape(N * wl // 16, 16)
              for c, wl in zip((corr_pyramid_0, corr_pyramid_1,
                                corr_pyramid_2, corr_pyramid_3), WLS)]

    mesh = plsc.VectorSubcoreMesh(core_axis_name="c", subcore_axis_name="s")
    scratch = (
        [pltpu.VMEM((PIX_PER_W,), jnp.float32)]
        + [pltpu.VMEM((2 * CHUNK, 16), jnp.float32) for _ in range(NUM_LEVELS)]
        + [pltpu.VMEM((CHUNK,), jnp.int32) for _ in range(2 * NUM_LEVELS)]
        + [pltpu.VMEM((NUM_LEVELS * K * CHUNK,), jnp.float32),
           pltpu.SemaphoreType.DMA, pltpu.SemaphoreType.DMA]
    )
    out = pl.kernel(
        _body,
        out_type=jax.ShapeDtypeStruct((B, NUM_LEVELS * K, HW), jnp.float32),
        mesh=mesh,
        compiler_params=pltpu.CompilerParams(needs_layout_passes=False,
                                             use_tc_tiling_on_sc=False),
        scratch_types=scratch,
    )(cc_flat, *corr2d)
    return out.reshape(B, NUM_LEVELS * K, H, W)


# speedup vs baseline: 3.5611x; 1.1997x over previous
import jax
import jax.numpy as jnp
from jax import lax
from jax.experimental import pallas as pl
from jax.experimental.pallas import tpu as pltpu, tpu_sc as plsc

RADIUS = 4
NUM_LEVELS = 4
K = 2 * RADIUS + 1

NC, NS, L = 2, 16, 16
NW = NC * NS

B, H, W = 8, 128, 160
HW = H * W
N = B * HW
PIX_PER_W = N // NW
CHUNK = 128
NCHUNK = PIX_PER_W // CHUNK
GRP = CHUNK // L
WLS = [W >> l for l in range(NUM_LEVELS)]
NROWS = [N * wl // 16 for wl in WLS]


def _body(cc_hbm, c0, c1, c2, c3, out_hbm,
          ccbuf,
          w0a, w1a, w2a, w3a, w0b, w1b, w2b, w3b,
          r00a, r10a, r01a, r11a, r02a, r12a, r03a, r13a,
          r00b, r10b, r01b, r11b, r02b, r12b, r03b, r13b,
          outa, outb, sga, sgb, soa, sob):
    corrs = [c0, c1, c2, c3]
    wins = [[w0a, w1a, w2a, w3a], [w0b, w1b, w2b, w3b]]
    r0s = [[r00a, r01a, r02a, r03a], [r00b, r01b, r02b, r03b]]
    r1s = [[r10a, r11a, r12a, r13a], [r10b, r11b, r12b, r13b]]
    outs = [outa, outb]
    gsems = [sga, sgb]
    osems = [soa, sob]

    wid = lax.axis_index("s") * NC + lax.axis_index("c")
    base_pix = wid * PIX_PER_W
    b = wid // 4
    hw_base = (wid % 4) * PIX_PER_W

    pltpu.sync_copy(cc_hbm.at[pl.ds(base_pix, PIX_PER_W)], ccbuf)

    lane = lax.iota(jnp.int32, L)

    def pass1(jc, par):
        chunk0 = jc * CHUNK
        def g_body(g, carry):
            off16 = g * L
            p_loc = off16 + lane
            cc = ccbuf[pl.ds(chunk0 + off16, L)]
            p_glob = base_pix + chunk0 + p_loc
            for l in range(NUM_LEVELS):
                c = cc * (0.5 ** l)
                i = c.astype(jnp.int32)
                s = p_glob * WLS[l] + (i - RADIUS)
                r0 = s >> 4
                r0s[par][l][pl.ds(off16, L)] = jnp.maximum(r0, 0)
                r1s[par][l][pl.ds(off16, L)] = jnp.minimum(r0 + 1, NROWS[l] - 1)
            return carry
        lax.fori_loop(0, GRP, g_body, 0)

    def gather_copies(par):
        return [pltpu.make_async_copy(
                    corrs[l].at[(r0s, r1s)[h][par][l]],
                    wins[par][l].at[pl.ds(h * CHUNK, CHUNK)], gsems[par])
                for l in range(NUM_LEVELS) for h in range(2)]

    def fire_gathers(par):
        for cp in gather_copies(par):
            cp.start()

    def wait_gathers(par):
        for cp in gather_copies(par):
            cp.wait()

    def out_copy(jc, par):
        return pltpu.make_async_copy(
            outs[par],
            out_hbm.at[b, :, pl.ds(hw_base + jc * CHUNK, CHUNK)], osems[par])

    def pass2(jc, par):
        chunk0 = jc * CHUNK
        def g_body(g, carry):
            off16 = g * L
            p_loc = off16 + lane
            cc = ccbuf[pl.ds(chunk0 + off16, L)]
            p_glob = base_pix + chunk0 + p_loc
            row_a = p_loc
            row_b = p_loc + CHUNK
            for l in range(NUM_LEVELS):
                wl = WLS[l]
                c = cc * (0.5 ** l)
                i = c.astype(jnp.int32)
                frac = c - i.astype(jnp.float32)
                omf = 1.0 - frac
                base = i - RADIUS
                s = p_glob * wl + base
                off = s & 15
                vs = []
                for m in range(K + 1):
                    q = off + m
                    row = jnp.where(q >= 16, row_b, row_a)
                    v = plsc.load_gather(wins[par][l], [row, q & 15])
                    if m < RADIUS:
                        v = jnp.where(base + m >= 0, v, 0.0)
                    elif m > RADIUS:
                        v = jnp.where(base + m < wl, v, 0.0)
                    vs.append(v)
                for k in range(K):
                    outs[par][l * K + k, pl.ds(off16, L)] = (
                        vs[k] * omf + vs[k + 1] * frac)
            return carry
        lax.fori_loop(0, GRP, g_body, 0)

    pass1(0, 0)
    fire_gathers(0)

    def pair_body(jj, carry):
        j0 = 2 * jj
        pass1(j0 + 1, 1)
        fire_gathers(1)
        wait_gathers(0)

        @pl.when(jj >= 1)
        def _():
            out_copy(j0 - 2, 0).wait()
        pass2(j0, 0)
        out_copy(j0, 0).start()

        @pl.when(jj < NCHUNK // 2 - 1)
        def _():
            pass1(j0 + 2, 0)
            fire_gathers(0)
        wait_gathers(1)

        @pl.when(jj >= 1)
        def _():
            out_copy(j0 - 1, 1).wait()
        pass2(j0 + 1, 1)
        out_copy(j0 + 1, 1).start()
        return carry

    lax.fori_loop(0, NCHUNK // 2, pair_body, 0)
    out_copy(NCHUNK - 2, 0).wait()
    out_copy(NCHUNK - 1, 1).wait()


@jax.jit
def kernel(centroids_coords, corr_pyramid_0, corr_pyramid_1, corr_pyramid_2,
           corr_pyramid_3):
    cc_flat = centroids_coords[:, 0].reshape(N)
    corr2d = [c.reshape(N * wl // 16, 16)
              for c, wl in zip((corr_pyramid_0, corr_pyramid_1,
                                corr_pyramid_2, corr_pyramid_3), WLS)]

    mesh = plsc.VectorSubcoreMesh(core_axis_name="c", subcore_axis_name="s")
    scratch = (
        [pltpu.VMEM((PIX_PER_W,), jnp.float32)]
        + [pltpu.VMEM((2 * CHUNK, 16), jnp.float32) for _ in range(8)]
        + [pltpu.VMEM((CHUNK,), jnp.int32) for _ in range(16)]
        + [pltpu.VMEM((NUM_LEVELS * K, CHUNK), jnp.float32) for _ in range(2)]
        + [pltpu.SemaphoreType.DMA for _ in range(4)]
    )
    out = pl.kernel(
        _body,
        out_type=jax.ShapeDtypeStruct((B, NUM_LEVELS * K, HW), jnp.float32),
        mesh=mesh,
        compiler_params=pltpu.CompilerParams(needs_layout_passes=False,
                                             use_tc_tiling_on_sc=False),
        scratch_types=scratch,
    )(cc_flat, *corr2d)
    return out.reshape(B, NUM_LEVELS * K, H, W)


# speedup vs baseline: 12.0343x; 3.3793x over previous
import jax
import jax.numpy as jnp
from jax import lax
from jax.experimental import pallas as pl
from jax.experimental.pallas import tpu as pltpu, tpu_sc as plsc

RADIUS = 4
NUM_LEVELS = 4
K = 2 * RADIUS + 1

NC, NS, L = 2, 16, 16
NW = NC * NS

B, H, W = 8, 128, 160
HW = H * W
N = B * HW
PIX_PER_W = N // NW
CHUNK = 128
NCHUNK = PIX_PER_W // CHUNK
GRP = CHUNK // L
WLS = [W >> l for l in range(NUM_LEVELS)]
XT = [(wl + 7) // 8 for wl in WLS]


def _body(cc_hbm, c0, c1, c2, c3, out_hbm,
          ccbuf,
          w0a, w1a, w2a, w3a, w0b, w1b, w2b, w3b,
          outa, outb, sga, sgb, soa, sob):
    corrs = [c0, c1, c2, c3]
    wins = [[w0a, w1a, w2a, w3a], [w0b, w1b, w2b, w3b]]
    outs = [outa, outb]
    gsems = [sga, sgb]
    osems = [soa, sob]

    wid = lax.axis_index("s") * NC + lax.axis_index("c")
    base_pix = wid * PIX_PER_W
    b = wid // 4
    hw_base = (wid % 4) * PIX_PER_W
    nblk0 = wid * NCHUNK

    pltpu.sync_copy(cc_hbm.at[pl.ds(base_pix, PIX_PER_W)], ccbuf)

    lane = lax.iota(jnp.int32, L)

    def level_copies(jc, par):
        nblk = nblk0 + jc
        return [pltpu.make_async_copy(
                    corrs[l].at[:, nblk], wins[par][l], gsems[par])
                for l in range(NUM_LEVELS)]

    def fire_levels(jc, par):
        for cp in level_copies(jc, par):
            cp.start()

    def wait_levels(jc, par):
        for cp in level_copies(jc, par):
            cp.wait()

    def out_copy(jc, par):
        return pltpu.make_async_copy(
            outs[par],
            out_hbm.at[b, :, pl.ds(hw_base + jc * CHUNK, CHUNK)], osems[par])

    def pass2(jc, par):
        chunk0 = jc * CHUNK
        def g_body(g, carry):
            off16 = g * L
            nvec = off16 + lane
            cc = ccbuf[pl.ds(chunk0 + off16, L)]
            for l in range(NUM_LEVELS):
                wl = WLS[l]
                c = cc * (0.5 ** l)
                i = c.astype(jnp.int32)
                frac = c - i.astype(jnp.float32)
                omf = 1.0 - frac
                base = i - RADIUS
                vs = []
                for m in range(K + 1):
                    x = base + m
                    if m < RADIUS:
                        xc = jnp.maximum(x, 0)
                    elif m > RADIUS:
                        xc = jnp.minimum(x, wl - 1)
                    else:
                        xc = x
                    v = plsc.load_gather(wins[par][l], [xc >> 3, xc & 7, nvec])
                    if m < RADIUS:
                        v = jnp.where(x >= 0, v, 0.0)
                    elif m > RADIUS:
                        v = jnp.where(x < wl, v, 0.0)
                    vs.append(v)
                for k in range(K):
                    outs[par][l * K + k, pl.ds(off16, L)] = (
                        vs[k] * omf + vs[k + 1] * frac)
            return carry
        lax.fori_loop(0, GRP, g_body, 0)

    fire_levels(0, 0)

    def pair_body(jj, carry):
        j0 = 2 * jj
        fire_levels(j0 + 1, 1)
        wait_levels(j0, 0)

        @pl.when(jj >= 1)
        def _():
            out_copy(j0 - 2, 0).wait()
        pass2(j0, 0)
        out_copy(j0, 0).start()

        @pl.when(jj < NCHUNK // 2 - 1)
        def _():
            fire_levels(j0 + 2, 0)
        wait_levels(j0 + 1, 1)

        @pl.when(jj >= 1)
        def _():
            out_copy(j0 - 1, 1).wait()
        pass2(j0 + 1, 1)
        out_copy(j0 + 1, 1).start()
        return carry

    lax.fori_loop(0, NCHUNK // 2, pair_body, 0)
    out_copy(NCHUNK - 2, 0).wait()
    out_copy(NCHUNK - 1, 1).wait()


@jax.jit
def kernel(centroids_coords, corr_pyramid_0, corr_pyramid_1, corr_pyramid_2,
           corr_pyramid_3):
    cc_flat = centroids_coords[:, 0].reshape(N)
    corr4d = []
    for corr, wl, xt in zip((corr_pyramid_0, corr_pyramid_1, corr_pyramid_2,
                             corr_pyramid_3), WLS, XT):
        t = corr.reshape(N, wl).transpose(1, 0)
        if xt * 8 != wl:
            t = jnp.pad(t, ((0, xt * 8 - wl), (0, 0)))
        corr4d.append(
            t.reshape(xt, 8, N // CHUNK, CHUNK).transpose(0, 2, 1, 3))

    mesh = plsc.VectorSubcoreMesh(core_axis_name="c", subcore_axis_name="s")
    scratch = (
        [pltpu.VMEM((PIX_PER_W,), jnp.float32)]
        + [pltpu.VMEM((xt, 8, CHUNK), jnp.float32) for _ in range(2)
           for xt in XT]
        + [pltpu.VMEM((NUM_LEVELS * K, CHUNK), jnp.float32) for _ in range(2)]
        + [pltpu.SemaphoreType.DMA for _ in range(4)]
    )
    out = pl.kernel(
        _body,
        out_type=jax.ShapeDtypeStruct((B, NUM_LEVELS * K, HW), jnp.float32),
        mesh=mesh,
        compiler_params=pltpu.CompilerParams(needs_layout_passes=False,
                                             use_tc_tiling_on_sc=False),
        scratch_types=scratch,
    )(cc_flat, *corr4d)
    return out.reshape(B, NUM_LEVELS * K, H, W)


# speedup vs baseline: 13.9206x; 1.1567x over previous
import jax
import jax.numpy as jnp
from jax import lax
from jax.experimental import pallas as pl
from jax.experimental.pallas import tpu as pltpu, tpu_sc as plsc

RADIUS = 4
NUM_LEVELS = 4
K = 2 * RADIUS + 1

NC, NS, L = 2, 16, 16
NW = NC * NS

B, H, W = 8, 128, 160
HW = H * W
N = B * HW
PIX_PER_W = N // NW
CHUNK = 128
NCHUNK = PIX_PER_W // CHUNK
GRP = CHUNK // L
ROWS_BLK = 8
CH_PER_BLK = ROWS_BLK * W // CHUNK
WLS = [W >> l for l in range(NUM_LEVELS)]
XT3 = [wl // 8 for wl in WLS[:3]]


def _body(cc_hbm, c0, c1, c2, c3, out_hbm,
          ccbuf,
          w0a, w1a, w2a, w3a, w0b, w1b, w2b, w3b,
          outT, sga, sgb, so):
    corrs = [c0, c1, c2, c3]
    wins = [[w0a, w1a, w2a, w3a], [w0b, w1b, w2b, w3b]]
    gsems = [sga, sgb]

    wid = lax.axis_index("s") * NC + lax.axis_index("c")
    b = wid // 4
    h0 = (wid % 4) * (H // 4)
    nblk0 = wid * NCHUNK

    pltpu.sync_copy(cc_hbm.at[b, 0, :, :, pl.ds(h0, H // 4)], ccbuf)

    lane = lax.iota(jnp.int32, L)

    def level_copies(jc, par):
        nblk = nblk0 + jc
        return [pltpu.make_async_copy(
                    corrs[l].at[:, nblk], wins[par][l], gsems[par])
                for l in range(NUM_LEVELS)]

    def fire_levels(jc, par):
        for cp in level_copies(jc, par):
            cp.start()

    def wait_levels(jc, par):
        for cp in level_copies(jc, par):
            cp.wait()

    def out_copy(blk):
        return pltpu.make_async_copy(
            outT, out_hbm.at[b, :, :, pl.ds(h0 + blk * ROWS_BLK, ROWS_BLK)],
            so)

    def pass2(jc, par):
        def g_body(g, carry):
            off16 = g * L
            nvec = off16 + lane
            p0 = jc * CHUNK + off16
            w0 = p0 % W
            dh = (p0 // W) % ROWS_BLK
            wvec = w0 + lane
            zero = 0 * lane
            cc = plsc.load_gather(ccbuf, [wvec >> 3, wvec & 7,
                                          (p0 // W) + zero])
            dhv = dh + zero
            for l in range(NUM_LEVELS):
                wl = WLS[l]
                c = cc * (0.5 ** l)
                i = c.astype(jnp.int32)
                frac = c - i.astype(jnp.float32)
                omf = 1.0 - frac
                base = i - RADIUS
                vs = []
                for m in range(K + 1):
                    x = base + m
                    if m < RADIUS:
                        xc = jnp.maximum(x, 0)
                    elif m > RADIUS:
                        xc = jnp.minimum(x, wl - 1)
                    else:
                        xc = x
                    if l < 3:
                        v = plsc.load_gather(wins[par][l],
                                             [xc >> 3, xc & 7, nvec])
                    else:
                        v = plsc.load_gather(wins[par][l], [xc, nvec])
                    if m < RADIUS:
                        v = jnp.where(x >= 0, v, 0.0)
                    elif m > RADIUS:
                        v = jnp.where(x < wl, v, 0.0)
                    vs.append(v)
                for k in range(K):
                    ch = l * K + k
                    plsc.store_scatter(outT, [ch + zero, wvec, dhv],
                                       vs[k] * omf + vs[k + 1] * frac)
            return carry
        lax.fori_loop(0, GRP, g_body, 0)

    fire_levels(0, 0)

    def pair_body(jj, carry):
        j0 = 2 * jj
        fire_levels(j0 + 1, 1)
        wait_levels(j0, 0)

        @pl.when((jj % (CH_PER_BLK // 2) == 0) & (jj >= CH_PER_BLK // 2))
        def _():
            out_copy(j0 // CH_PER_BLK - 1).wait()
        pass2(j0, 0)

        @pl.when(jj < NCHUNK // 2 - 1)
        def _():
            fire_levels(j0 + 2, 0)
        wait_levels(j0 + 1, 1)
        pass2(j0 + 1, 1)

        @pl.when(jj % (CH_PER_BLK // 2) == CH_PER_BLK // 2 - 1)
        def _():
            out_copy(j0 // CH_PER_BLK).start()
        return carry

    lax.fori_loop(0, NCHUNK // 2, pair_body, 0)
    out_copy(NCHUNK // CH_PER_BLK - 1).wait()


@jax.jit
def kernel(centroids_coords, corr_pyramid_0, corr_pyramid_1, corr_pyramid_2,
           corr_pyramid_3):
    cc6 = centroids_coords.transpose(0, 1, 3, 2).reshape(B, 2, W // 8, 8, H)
    corr4d = []
    for corr, wl in zip((corr_pyramid_0, corr_pyramid_1, corr_pyramid_2),
                        WLS[:3]):
        t = corr.reshape(N, wl).transpose(1, 0)
        corr4d.append(
            t.reshape(wl // 8, 8, N // CHUNK, CHUNK).transpose(0, 2, 1, 3))
    c3 = corr_pyramid_3.reshape(N, WLS[3]).transpose(1, 0).reshape(
        WLS[3], N // CHUNK, CHUNK)

    mesh = plsc.VectorSubcoreMesh(core_axis_name="c", subcore_axis_name="s")
    scratch = (
        [pltpu.VMEM((W // 8, 8, H // 4), jnp.float32)]
        + [pltpu.VMEM(shp, jnp.float32) for _ in range(2)
           for shp in [(XT3[0], 8, CHUNK), (XT3[1], 8, CHUNK),
                       (XT3[2], 8, CHUNK), (WLS[3], CHUNK)]]
        + [pltpu.VMEM((NUM_LEVELS * K, W, ROWS_BLK), jnp.float32)]
        + [pltpu.SemaphoreType.DMA for _ in range(3)]
    )
    out = pl.kernel(
        _body,
        out_type=jax.ShapeDtypeStruct((B, NUM_LEVELS * K, W, H), jnp.float32),
        mesh=mesh,
        compiler_params=pltpu.CompilerParams(needs_layout_passes=False,
                                             use_tc_tiling_on_sc=False),
        scratch_types=scratch,
    )(cc6, *corr4d, c3)
    return out.transpose(0, 1, 3, 2)


# speedup vs baseline: 14.5745x; 1.0470x over previous
import jax
import jax.numpy as jnp
from jax import lax
from jax.experimental import pallas as pl
from jax.experimental.pallas import tpu as pltpu, tpu_sc as plsc

RADIUS = 4
NUM_LEVELS = 4
K = 2 * RADIUS + 1

NC, NS, L = 2, 16, 16
NW = NC * NS

B, H, W = 8, 128, 160
HW = H * W
N = B * HW
PIX_PER_W = N // NW
CHUNK = 128
NCHUNK = PIX_PER_W // CHUNK
GRP = CHUNK // L
ROWS_BLK = 8
CH_PER_BLK = ROWS_BLK * W // CHUNK
WLS = [W >> l for l in range(NUM_LEVELS)]
XT3 = [wl // 8 for wl in WLS[:3]]


def _body(cc_hbm, c0, c1, c2, c3, out_hbm,
          ccbuf,
          w0a, w1a, w2a, w3a, w0b, w1b, w2b, w3b,
          outT, sga, sgb, so):
    corrs = [c0, c1, c2, c3]
    wins = [[w0a, w1a, w2a, w3a], [w0b, w1b, w2b, w3b]]
    gsems = [sga, sgb]

    wid = lax.axis_index("s") * NC + lax.axis_index("c")
    b = wid // 4
    h0 = (wid % 4) * (H // 4)
    nblk0 = wid * NCHUNK

    pltpu.sync_copy(cc_hbm.at[b, 0, :, :, pl.ds(h0, H // 4)], ccbuf)

    lane = lax.iota(jnp.int32, L)

    def level_copies(jc, par):
        nblk = nblk0 + jc
        return [pltpu.make_async_copy(
                    corrs[l].at[:, nblk], wins[par][l], gsems[par])
                for l in range(NUM_LEVELS)]

    def fire_levels(jc, par):
        for cp in level_copies(jc, par):
            cp.start()

    def wait_levels(jc, par):
        for cp in level_copies(jc, par):
            cp.wait()

    def out_copy(blk):
        return pltpu.make_async_copy(
            outT, out_hbm.at[b, :, :, pl.ds(h0 + blk * ROWS_BLK, ROWS_BLK)],
            so)

    def pass2(jc, par):
        def g_body(g, carry):
            off16 = g * L
            nvec = off16 + lane
            p0 = jc * CHUNK + off16
            w0 = p0 % W
            dh = (p0 // W) % ROWS_BLK
            wvec = w0 + lane
            zero = 0 * lane
            cc = plsc.load_gather(ccbuf, [wvec >> 3, wvec & 7,
                                          (p0 // W) + zero])
            dhv = dh + zero
            for l in range(NUM_LEVELS):
                wl = WLS[l]
                c = cc * (0.5 ** l)
                i = c.astype(jnp.int32)
                frac = c - i.astype(jnp.float32)
                omf = 1.0 - frac
                base = i - RADIUS
                vs = []
                for m in range(K + 1):
                    x = base + m
                    if m < RADIUS:
                        xc = jnp.maximum(x, 0)
                    elif m > RADIUS:
                        xc = jnp.minimum(x, wl - 1)
                    else:
                        xc = x
                    v = plsc.load_gather(wins[par][l],
                                         [xc >> 3, xc & 7, nvec])
                    if m < RADIUS:
                        v = jnp.where(x >= 0, v, 0.0)
                    elif m > RADIUS:
                        v = jnp.where(x < wl, v, 0.0)
                    vs.append(v)
                for k in range(K):
                    ch = l * K + k
                    plsc.store_scatter(outT, [ch + zero, wvec, dhv],
                                       vs[k] * omf + vs[k + 1] * frac)
            return carry
        lax.fori_loop(0, GRP, g_body, 0)

    fire_levels(0, 0)

    def pair_body(jj, carry):
        j0 = 2 * jj
        fire_levels(j0 + 1, 1)
        wait_levels(j0, 0)

        @pl.when((jj % (CH_PER_BLK // 2) == 0) & (jj >= CH_PER_BLK // 2))
        def _():
            out_copy(j0 // CH_PER_BLK - 1).wait()
        pass2(j0, 0)

        @pl.when(jj < NCHUNK // 2 - 1)
        def _():
            fire_levels(j0 + 2, 0)
        wait_levels(j0 + 1, 1)
        pass2(j0 + 1, 1)

        @pl.when(jj % (CH_PER_BLK // 2) == CH_PER_BLK // 2 - 1)
        def _():
            out_copy(j0 // CH_PER_BLK).start()
        return carry

    lax.fori_loop(0, NCHUNK // 2, pair_body, 0)
    out_copy(NCHUNK // CH_PER_BLK - 1).wait()


@jax.jit
def kernel(centroids_coords, corr_pyramid_0, corr_pyramid_1, corr_pyramid_2,
           corr_pyramid_3):
    cc6 = centroids_coords.transpose(0, 1, 3, 2).reshape(B, 2, W // 8, 8, H)
    corr4d = []
    for corr, wl in zip((corr_pyramid_0, corr_pyramid_1, corr_pyramid_2),
                        WLS[:3]):
        t = corr.reshape(N, wl).transpose(1, 0)
        corr4d.append(
            t.reshape(wl // 8, 8, N // CHUNK, CHUNK).transpose(0, 2, 1, 3))
    t3 = corr_pyramid_3.reshape(N, WLS[3]).transpose(1, 0)
    c3 = jnp.pad(t3, ((0, 4), (0, 0))).reshape(
        3, 8, N // CHUNK, CHUNK).transpose(0, 2, 1, 3)

    mesh = plsc.VectorSubcoreMesh(core_axis_name="c", subcore_axis_name="s")
    scratch = (
        [pltpu.VMEM((W // 8, 8, H // 4), jnp.float32)]
        + [pltpu.VMEM(shp, jnp.float32) for _ in range(2)
           for shp in [(XT3[0], 8, CHUNK), (XT3[1], 8, CHUNK),
                       (XT3[2], 8, CHUNK), (3, 8, CHUNK)]]
        + [pltpu.VMEM((NUM_LEVELS * K, W, ROWS_BLK), jnp.float32)]
        + [pltpu.SemaphoreType.DMA for _ in range(3)]
    )
    out = pl.kernel(
        _body,
        out_type=jax.ShapeDtypeStruct((B, NUM_LEVELS * K, W, H), jnp.float32),
        mesh=mesh,
        compiler_params=pltpu.CompilerParams(needs_layout_passes=False,
                                             use_tc_tiling_on_sc=False),
        scratch_types=scratch,
    )(cc6, *corr4d, c3)
    return out.transpose(0, 1, 3, 2)
